# baseline (speedup 1.0000x reference)
import functools

import jax
import jax.numpy as jnp
from jax import lax
from jax.experimental import pallas as pl
from jax.experimental.pallas import tpu as pltpu
from jax.experimental.pallas import tpu_sc as plsc

G = 256
CHUNK = 128
NBUF = 4
NC, NS = 2, 16
NW = NC * NS


def _sc_edge_agg(hm, src3, dst3, zrows, nacc):
    n2, hh = hm.shape
    h = NC * hh
    nch = src3.shape[2]
    rpt = nacc // NS
    mesh = plsc.VectorSubcoreMesh(core_axis_name="c", subcore_axis_name="s")

    @functools.partial(
        pl.kernel,
        out_type=jax.ShapeDtypeStruct((nacc, h), jnp.bfloat16),
        mesh=mesh,
        compiler_params=pltpu.CompilerParams(use_tc_tiling_on_sc=False),
        scratch_types=[
            pltpu.VMEM((nch, CHUNK), jnp.int32),
            pltpu.VMEM((nch, CHUNK), jnp.int32),
            pltpu.VMEM((NBUF, CHUNK, hh), jnp.bfloat16),
            pltpu.VMEM_SHARED((nacc, hh), jnp.bfloat16),
            pltpu.SemaphoreType.DMA,
            pltpu.SemaphoreType.DMA,
            pltpu.SemaphoreType.DMA,
            pltpu.SemaphoreType.DMA,
        ],
    )
    def k(hm_hbm, src_hbm, dst_hbm, zrows_hbm, parts_hbm, sidx, didx, rows,
          acc, s0, s1, s2, s3):
        cid = lax.axis_index("c")
        sid = lax.axis_index("s")
        sems = [s0, s1, s2, s3]
        table = hm_hbm

        pltpu.sync_copy(src_hbm.at[cid, sid], sidx)
        pltpu.sync_copy(dst_hbm.at[sid], didx)
        pltpu.sync_copy(zrows_hbm, acc.at[pl.ds(sid * rpt, rpt)])
        for b in range(NBUF):
            pltpu.async_copy(table.at[sidx.at[b]], rows.at[b], sems[b])
        plsc.subcore_barrier()

        def grp(g, carry):
            for b in range(NBUF):
                j = g * NBUF + b
                pltpu.make_async_copy(
                    table.at[pl.ds(0, CHUNK)], rows.at[b], sems[b]
                ).wait()
                pltpu.sync_copy(rows.at[b], acc.at[didx.at[j]], add=True)

                @pl.when(j + NBUF < nch)
                def _():
                    pltpu.async_copy(
                        table.at[sidx.at[j + NBUF]], rows.at[b], sems[b]
                    )
            return carry

        lax.fori_loop(0, nch // NBUF, grp, 0)
        plsc.subcore_barrier()
        pltpu.sync_copy(
            acc.at[pl.ds(sid * rpt, rpt)],
            parts_hbm.at[pl.ds(sid * rpt, rpt), pl.ds(cid * hh, hh)],
        )

    return k(hm, src3, dst3, zrows)


def _tc_prep(eidx, epad, n):
    e = eidx.shape[1]

    def body(e_ref, s_ref, d_ref):
        ei = e_ref[...]
        s2 = ei[0:1, :] * 2
        s_ref[0:1, :e] = s2
        s_ref[1:2, :e] = s2 + 1
        d_ref[0:1, :e] = ei[1:2, :]
        pad = lax.broadcasted_iota(jnp.int32, (1, epad - e), 1)
        ps = (pad & 4095) * 2
        s_ref[0:1, e:] = ps
        s_ref[1:2, e:] = ps + 1
        d_ref[0:1, e:] = n + (pad & 63)

    return pl.pallas_call(
        body,
        out_shape=[
            jax.ShapeDtypeStruct((2, epad), jnp.int32),
            jax.ShapeDtypeStruct((1, epad), jnp.int32),
        ],
    )(eidx)


def _tc_init(nf, w_init, wm0, blk):
    n, d = nf.shape
    h = w_init.shape[1]

    def body(nf_ref, wi_ref, wm_ref, h_ref, hm_ref):
        hv = jnp.dot(nf_ref[...], wi_ref[...], preferred_element_type=jnp.float32)
        h_ref[...] = hv
        hm_ref[...] = jnp.dot(hv, wm_ref[...],
                              preferred_element_type=jnp.float32
                              ).astype(jnp.bfloat16)

    return pl.pallas_call(
        body,
        grid=(n // blk,),
        in_specs=[
            pl.BlockSpec((blk, d), lambda i: (i, 0)),
            pl.BlockSpec((d, h), lambda i: (0, 0)),
            pl.BlockSpec((h, h), lambda i: (0, 0)),
        ],
        out_specs=[pl.BlockSpec((blk, h), lambda i: (i, 0))] * 2,
        out_shape=[
            jax.ShapeDtypeStruct((n, h), jnp.float32),
            jax.ShapeDtypeStruct((n, h), jnp.bfloat16),
        ],
    )(nf, w_init, wm0)


def _tc_layer(hs, parts, w_self, wm_next, blk):
    n, h = hs.shape
    last = wm_next is None

    def body(h_ref, p_ref, ws_ref, *rest):
        hv = h_ref[...]
        agg = p_ref[...].astype(jnp.float32)
        hn = jax.nn.relu(agg + jnp.dot(hv, ws_ref[...],
                                       preferred_element_type=jnp.float32)) + hv
        if last:
            (hn_ref,) = rest
            hn_ref[...] = hn
        else:
            wm_ref, hn_ref, hm_ref = rest
            hn_ref[...] = hn
            hm_ref[...] = jnp.dot(hn, wm_ref[...],
                                  preferred_element_type=jnp.float32
                                  ).astype(jnp.bfloat16)

    in_specs = [
        pl.BlockSpec((blk, h), lambda i: (i, 0)),
        pl.BlockSpec((blk, h), lambda i: (i, 0)),
        pl.BlockSpec((h, h), lambda i: (0, 0)),
    ]
    ins = [hs, parts, w_self]
    out_specs = [pl.BlockSpec((blk, h), lambda i: (i, 0))]
    out_shape = [jax.ShapeDtypeStruct((n, h), jnp.float32)]
    if not last:
        in_specs.append(pl.BlockSpec((h, h), lambda i: (0, 0)))
        ins.append(wm_next)
        out_specs.append(pl.BlockSpec((blk, h), lambda i: (i, 0)))
        out_shape.append(jax.ShapeDtypeStruct((n, h), jnp.bfloat16))
    return pl.pallas_call(
        body,
        grid=(n // blk,),
        in_specs=in_specs,
        out_specs=out_specs,
        out_shape=out_shape,
    )(*ins)


def _tc_pool_state(n2g3, state, blk, with_counts):
    n, h = state.shape

    def body(n2g_ref, s_ref, *orefs):
        ids = n2g_ref[0, :, :]
        iota = lax.broadcasted_iota(jnp.int32, (G, blk), 0)
        mask = (ids == iota).astype(jnp.float32)
        pooled = jnp.dot(mask, s_ref[...], preferred_element_type=jnp.float32)

        @pl.when(pl.program_id(0) == 0)
        def _():
            for o in orefs:
                o[...] = jnp.zeros_like(o)

        orefs[0][...] += pooled
        if with_counts:
            cnt = jnp.sum(mask, axis=1, keepdims=True)
            orefs[1][...] += jnp.broadcast_to(cnt, (G, h))

    nout = 2 if with_counts else 1
    out_specs = [pl.BlockSpec((G, h), lambda i: (0, 0))] * nout
    out_shape = [jax.ShapeDtypeStruct((G, h), jnp.float32)] * nout
    res = pl.pallas_call(
        body,
        grid=(n // blk,),
        in_specs=[
            pl.BlockSpec((1, 1, blk), lambda i: (i, 0, 0)),
            pl.BlockSpec((blk, h), lambda i: (i, 0)),
        ],
        out_specs=out_specs,
        out_shape=out_shape,
    )(n2g3, state)
    return res if with_counts else res[0]


def _tc_project(pooleds, counts, w_ro):
    rep, out = w_ro.shape
    ns = len(pooleds)
    h = rep // ns

    def body(*refs):
        c_ref, w_ref, o_ref = refs[ns], refs[ns + 1], refs[ns + 2]
        inv = 1.0 / jnp.maximum(c_ref[...][:, :1], 1.0)
        p = jnp.concatenate([refs[s][...] for s in range(ns)], axis=1) * inv
        o_ref[...] = jnp.dot(p, w_ref[...], preferred_element_type=jnp.float32)

    return pl.pallas_call(
        body,
        out_shape=jax.ShapeDtypeStruct((G, out), jnp.float32),
    )(*pooleds, counts, w_ro)


def kernel(node_features, edge_index, node_to_graph, num_graphs, W_init,
           W_msg, W_self, W_ro):
    n, _ = node_features.shape
    e = edge_index.shape[1]
    hdim = W_init.shape[1]
    nlayers = W_msg.shape[0]
    blk = 10000

    quant = NS * NBUF * CHUNK
    epad = -(-e // quant) * quant
    nacc = -(-(n + 1) // (NS * 8)) * (NS * 8)
    src6, dst6 = _tc_prep(edge_index, epad, n)
    src3 = src6.reshape(NC, NS, epad // (NS * CHUNK), CHUNK)
    dst3 = dst6.reshape(NS, epad // (NS * CHUNK), CHUNK)
    zrows = jnp.zeros((nacc // NS, hdim // 2), jnp.bfloat16)
    n2g3 = node_to_graph.reshape(n // blk, 1, blk)

    h, hm = _tc_init(node_features, W_init, W_msg[0], blk)
    pooled0, counts = _tc_pool_state(n2g3, h, blk, with_counts=True)
    pooleds = [pooled0]
    for l in range(nlayers):
        parts = _sc_edge_agg(hm.reshape(2 * n, hdim // 2), src3, dst3,
                             zrows, nacc)
        wm_next = W_msg[l + 1] if l + 1 < nlayers else None
        res = _tc_layer(h, parts, W_self[l], wm_next, blk)
        if wm_next is None:
            (h,) = res
        else:
            h, hm = res
        pooleds.append(_tc_pool_state(n2g3, h, blk, with_counts=False))

    out = _tc_project(pooleds, counts, W_ro)
    return jnp.where(jnp.arange(G)[:, None] < num_graphs, out, 0.0)


# speedup vs baseline: 1.0045x; 1.0045x over previous
import functools

import jax
import jax.numpy as jnp
from jax import lax
from jax.experimental import pallas as pl
from jax.experimental.pallas import tpu as pltpu
from jax.experimental.pallas import tpu_sc as plsc

G = 256
CHUNK = 128
NBUF = 4
NC, NS = 2, 16
NW = NC * NS


def _sc_edge_agg(hm, src3, dst3, zrows, nacc):
    n2, hh = hm.shape
    h = NC * hh
    nch = src3.shape[2]
    rpt = nacc // NS
    mesh = plsc.VectorSubcoreMesh(core_axis_name="c", subcore_axis_name="s")

    @functools.partial(
        pl.kernel,
        out_type=jax.ShapeDtypeStruct((nacc, h), jnp.bfloat16),
        mesh=mesh,
        compiler_params=pltpu.CompilerParams(use_tc_tiling_on_sc=False),
        scratch_types=[
            pltpu.VMEM((nch, CHUNK), jnp.int32),
            pltpu.VMEM((nch, CHUNK), jnp.int32),
            pltpu.VMEM((NBUF, CHUNK, hh), jnp.bfloat16),
            pltpu.VMEM_SHARED((nacc, hh), jnp.bfloat16),
            pltpu.SemaphoreType.DMA,
            pltpu.SemaphoreType.DMA,
            pltpu.SemaphoreType.DMA,
            pltpu.SemaphoreType.DMA,
        ],
    )
    def k(hm_hbm, src_hbm, dst_hbm, zrows_hbm, parts_hbm, sidx, didx, rows,
          acc, s0, s1, s2, s3):
        cid = lax.axis_index("c")
        sid = lax.axis_index("s")
        sems = [s0, s1, s2, s3]
        table = hm_hbm

        pltpu.sync_copy(src_hbm.at[cid, sid], sidx)
        pltpu.sync_copy(dst_hbm.at[sid], didx)
        pltpu.sync_copy(zrows_hbm, acc.at[pl.ds(sid * rpt, rpt)])
        for b in range(NBUF):
            pltpu.async_copy(table.at[sidx.at[b]], rows.at[b], sems[b])
        plsc.subcore_barrier()

        def grp(g, carry):
            for b in range(NBUF):
                j = g * NBUF + b
                pltpu.make_async_copy(
                    table.at[pl.ds(0, CHUNK)], rows.at[b], sems[b]
                ).wait()
                pltpu.sync_copy(rows.at[b], acc.at[didx.at[j]], add=True)

                @pl.when(j + NBUF < nch)
                def _():
                    pltpu.async_copy(
                        table.at[sidx.at[j + NBUF]], rows.at[b], sems[b]
                    )
            return carry

        lax.fori_loop(0, nch // NBUF, grp, 0)
        plsc.subcore_barrier()
        pltpu.sync_copy(
            acc.at[pl.ds(sid * rpt, rpt)],
            parts_hbm.at[pl.ds(sid * rpt, rpt), pl.ds(cid * hh, hh)],
        )

    return k(hm, src3, dst3, zrows)


def _tc_prep(eidx, epad, n):
    e = eidx.shape[1]

    def body(e_ref, s_ref, d_ref):
        ei = e_ref[...]
        s2 = ei[0:1, :] * 2
        s_ref[0:1, :e] = s2
        s_ref[1:2, :e] = s2 + 1
        d_ref[0:1, :e] = ei[1:2, :]
        pad = lax.broadcasted_iota(jnp.int32, (1, epad - e), 1)
        ps = (pad & 4095) * 2
        s_ref[0:1, e:] = ps
        s_ref[1:2, e:] = ps + 1
        d_ref[0:1, e:] = n + (pad & 63)

    return pl.pallas_call(
        body,
        out_shape=[
            jax.ShapeDtypeStruct((2, epad), jnp.int32),
            jax.ShapeDtypeStruct((1, epad), jnp.int32),
        ],
    )(eidx)


def _tc_init(nf, w_init, wm0, blk):
    n, d = nf.shape
    h = w_init.shape[1]

    def body(nf_ref, wi_ref, wm_ref, h_ref, hm_ref):
        hv = jnp.dot(nf_ref[...], wi_ref[...], preferred_element_type=jnp.float32)
        h_ref[...] = hv
        hm_ref[...] = jnp.dot(hv, wm_ref[...],
                              preferred_element_type=jnp.float32
                              ).astype(jnp.bfloat16)

    return pl.pallas_call(
        body,
        grid=(n // blk,),
        in_specs=[
            pl.BlockSpec((blk, d), lambda i: (i, 0)),
            pl.BlockSpec((d, h), lambda i: (0, 0)),
            pl.BlockSpec((h, h), lambda i: (0, 0)),
        ],
        out_specs=[pl.BlockSpec((blk, h), lambda i: (i, 0))] * 2,
        out_shape=[
            jax.ShapeDtypeStruct((n, h), jnp.float32),
            jax.ShapeDtypeStruct((n, h), jnp.bfloat16),
        ],
    )(nf, w_init, wm0)


def _tc_layer(hs, parts, w_self, wm_next, blk):
    n, h = hs.shape
    last = wm_next is None

    def body(h_ref, p_ref, ws_ref, *rest):
        hv = h_ref[...]
        agg = p_ref[...].astype(jnp.float32)
        hn = jax.nn.relu(agg + jnp.dot(hv, ws_ref[...],
                                       preferred_element_type=jnp.float32)) + hv
        if last:
            (hn_ref,) = rest
            hn_ref[...] = hn
        else:
            wm_ref, hn_ref, hm_ref = rest
            hn_ref[...] = hn
            hm_ref[...] = jnp.dot(hn, wm_ref[...],
                                  preferred_element_type=jnp.float32
                                  ).astype(jnp.bfloat16)

    in_specs = [
        pl.BlockSpec((blk, h), lambda i: (i, 0)),
        pl.BlockSpec((blk, h), lambda i: (i, 0)),
        pl.BlockSpec((h, h), lambda i: (0, 0)),
    ]
    ins = [hs, parts, w_self]
    out_specs = [pl.BlockSpec((blk, h), lambda i: (i, 0))]
    out_shape = [jax.ShapeDtypeStruct((n, h), jnp.float32)]
    if not last:
        in_specs.append(pl.BlockSpec((h, h), lambda i: (0, 0)))
        ins.append(wm_next)
        out_specs.append(pl.BlockSpec((blk, h), lambda i: (i, 0)))
        out_shape.append(jax.ShapeDtypeStruct((n, h), jnp.bfloat16))
    return pl.pallas_call(
        body,
        grid=(n // blk,),
        in_specs=in_specs,
        out_specs=out_specs,
        out_shape=out_shape,
    )(*ins)


def _tc_pool_state(n2g3, state, blk, with_counts):
    n, h = state.shape

    def body(n2g_ref, s_ref, *orefs):
        ids = n2g_ref[0, :, :]
        iota = lax.broadcasted_iota(jnp.int32, (G, blk), 0)
        mask = (ids == iota).astype(jnp.float32)
        pooled = jnp.dot(mask, s_ref[...], preferred_element_type=jnp.float32)

        @pl.when(pl.program_id(0) == 0)
        def _():
            for o in orefs:
                o[...] = jnp.zeros_like(o)

        orefs[0][...] += pooled
        if with_counts:
            cnt = jnp.sum(mask, axis=1, keepdims=True)
            orefs[1][...] += jnp.broadcast_to(cnt, (G, h))

    nout = 2 if with_counts else 1
    out_specs = [pl.BlockSpec((G, h), lambda i: (0, 0))] * nout
    out_shape = [jax.ShapeDtypeStruct((G, h), jnp.float32)] * nout
    res = pl.pallas_call(
        body,
        grid=(n // blk,),
        in_specs=[
            pl.BlockSpec((1, 1, blk), lambda i: (i, 0, 0)),
            pl.BlockSpec((blk, h), lambda i: (i, 0)),
        ],
        out_specs=out_specs,
        out_shape=out_shape,
    )(n2g3, state)
    return res if with_counts else res[0]


def _tc_project(pooleds, counts, w_ro):
    rep, out = w_ro.shape
    ns = len(pooleds)
    h = rep // ns

    def body(*refs):
        c_ref, w_ref, o_ref = refs[ns], refs[ns + 1], refs[ns + 2]
        inv = 1.0 / jnp.maximum(c_ref[...][:, :1], 1.0)
        p = jnp.concatenate([refs[s][...] for s in range(ns)], axis=1) * inv
        o_ref[...] = jnp.dot(p, w_ref[...], preferred_element_type=jnp.float32)

    return pl.pallas_call(
        body,
        out_shape=jax.ShapeDtypeStruct((G, out), jnp.float32),
    )(*pooleds, counts, w_ro)


def kernel(node_features, edge_index, node_to_graph, num_graphs, W_init,
           W_msg, W_self, W_ro):
    n, _ = node_features.shape
    e = edge_index.shape[1]
    hdim = W_init.shape[1]
    nlayers = W_msg.shape[0]
    blk = 5000

    quant = NS * NBUF * CHUNK
    epad = -(-e // quant) * quant
    nacc = -(-(n + 1) // (NS * 8)) * (NS * 8)
    src6, dst6 = _tc_prep(edge_index, epad, n)
    src3 = src6.reshape(NC, NS, epad // (NS * CHUNK), CHUNK)
    dst3 = dst6.reshape(NS, epad // (NS * CHUNK), CHUNK)
    zrows = jnp.zeros((nacc // NS, hdim // 2), jnp.bfloat16)
    n2g3 = node_to_graph.reshape(n // blk, 1, blk)

    h, hm = _tc_init(node_features, W_init, W_msg[0], blk)
    pooled0, counts = _tc_pool_state(n2g3, h, blk, with_counts=True)
    pooleds = [pooled0]
    for l in range(nlayers):
        parts = _sc_edge_agg(hm.reshape(2 * n, hdim // 2), src3, dst3,
                             zrows, nacc)
        wm_next = W_msg[l + 1] if l + 1 < nlayers else None
        res = _tc_layer(h, parts, W_self[l], wm_next, blk)
        if wm_next is None:
            (h,) = res
        else:
            h, hm = res
        pooleds.append(_tc_pool_state(n2g3, h, blk, with_counts=False))

    out = _tc_project(pooleds, counts, W_ro)
    return jnp.where(jnp.arange(G)[:, None] < num_graphs, out, 0.0)
